# baseline (speedup 1.0000x reference)
import functools

import jax
import jax.numpy as jnp
from jax.experimental import pallas as pl
from jax.experimental.pallas import tpu as pltpu


def _round_up(v, m):
    return ((v + m - 1) // m) * m


def _tree_reduce(vals, op):
    vals = list(vals)
    while len(vals) > 1:
        nxt = [op(vals[i], vals[i + 1]) for i in range(0, len(vals) - 1, 2)]
        if len(vals) % 2:
            nxt.append(vals[-1])
        vals = nxt
    return vals[0]


def _toeplitz(wrow, n):
    k = wrow.shape[0]
    p = k // 2
    m = jnp.zeros((n, n), jnp.float32)
    for d in range(-p, p + 1):
        m = m + wrow[d + p] * jnp.eye(n, n, k=-d, dtype=jnp.float32)
    return m


def _build_conv_mats(w0, w1c, wth, kk):
    p = kk // 2
    mats = jnp.zeros((5, 4 * wth, 2 * wth), jnp.float32)
    for dlt in range(-2, 3):
        for pp in (0, 1):
            for q in (0, 1):
                ki = 2 * dlt + pp - q + p
                if 0 <= ki < kk:
                    t0 = _toeplitz(w0[ki], wth)
                    t1 = _toeplitz(w1c[ki], wth)
                    r0 = pp * wth
                    c0 = q * wth
                    mats = mats.at[dlt + 2, r0:r0 + wth, c0:c0 + wth].add(t0)
                    mats = mats.at[dlt + 2, 2 * wth + r0:2 * wth + r0 + wth,
                                   c0:c0 + wth].add(t1)
    return mats


def _sa_body(x_ref, m_ref, corr_ref, s_ref, o_ref, pad_ref, *,
             C, HPAIR, G, UNROLL):
    w2 = x_ref.shape[2]
    step_rows = G * HPAIR

    def body(i, carry):
        m, s = carry
        base = pl.multiple_of(i * step_rows, step_rows)
        cs = [x_ref[0, pl.ds(base + k * HPAIR, HPAIR), :] for k in range(G)]
        m1 = _tree_reduce(cs, jnp.maximum)
        s1 = _tree_reduce(cs, jnp.add)
        return jnp.maximum(m, m1), s + s1

    init = (jnp.full((HPAIR, w2), -jnp.inf, jnp.float32),
            jnp.zeros((HPAIR, w2), jnp.float32))
    mx, sm = jax.lax.fori_loop(0, C // G, body, init, unroll=UNROLL)

    c0 = jnp.mean(mx)
    c1 = jnp.mean(sm)

    pad_ref[0:2, :] = jnp.zeros((2, 2 * w2), jnp.float32)
    pad_ref[pl.ds(2 + HPAIR, 2), :] = jnp.zeros((2, 2 * w2), jnp.float32)
    pad_ref[pl.ds(2, HPAIR), 0:w2] = mx - c0
    pad_ref[pl.ds(2, HPAIR), w2:2 * w2] = sm - c1

    acc = None
    for dlt in range(5):
        win = pad_ref[pl.ds(dlt, HPAIR), :]
        mm = jnp.dot(win, m_ref[dlt], preferred_element_type=jnp.float32)
        acc = mm if acc is None else acc + mm

    z = acc + c0 * corr_ref[0] + c1 * corr_ref[1] + s_ref[0]
    o_ref[0, 0] = jax.nn.sigmoid(z).astype(o_ref.dtype)


def _spatial_attention(x, weight, bias):
    B, C, H, W = x.shape
    kk = weight.shape[2]
    p = kk // 2
    assert H % 2 == 0 and W == 64 and C % 8 == 0
    hpair = H // 2
    w2 = 2 * W

    x3 = x.reshape(B, C * hpair, w2)

    w0 = weight[0, 0].astype(jnp.float32)
    w1c = weight[0, 1].astype(jnp.float32) * (1.0 / C)
    mats = _build_conv_mats(w0, w1c, W, kk)

    hh = jnp.arange(H)[:, None] + jnp.arange(kk)[None, :] - p
    um = ((hh >= 0) & (hh < H)).astype(jnp.float32)
    wwv = jnp.arange(W)[:, None] + jnp.arange(kk)[None, :] - p
    vm = ((wwv >= 0) & (wwv < W)).astype(jnp.float32)
    s0 = um @ w0 @ vm.T
    s1 = um @ w1c @ vm.T
    corr = jnp.stack([s0.reshape(hpair, w2), s1.reshape(hpair, w2)])

    bias_s = bias.reshape(-1).astype(jnp.float32)

    pad_rows = _round_up(hpair + 4, 8)
    G = 8
    body = functools.partial(_sa_body, C=C, HPAIR=hpair, G=G, UNROLL=2)

    cost = pl.CostEstimate(
        flops=int(B * H * W * (2 * C + 4 * kk * kk + 4)),
        transcendentals=int(B * H * W),
        bytes_accessed=int(B * (C + 1) * H * W * 4 + mats.size * 4),
    )

    out = pl.pallas_call(
        body,
        out_shape=jax.ShapeDtypeStruct((B, 1, hpair, w2), x.dtype),
        grid=(B,),
        in_specs=[
            pl.BlockSpec((1, C * hpair, w2), lambda b: (b, 0, 0)),
            pl.BlockSpec((5, 4 * W, w2), lambda b: (0, 0, 0)),
            pl.BlockSpec((2, hpair, w2), lambda b: (0, 0, 0)),
            pl.BlockSpec(memory_space=pltpu.MemorySpace.SMEM),
        ],
        out_specs=pl.BlockSpec((1, 1, hpair, w2), lambda b: (b, 0, 0, 0)),
        scratch_shapes=[
            pltpu.VMEM((pad_rows, 2 * w2), jnp.float32),
        ],
        compiler_params=pltpu.CompilerParams(
            dimension_semantics=("parallel",),
            vmem_limit_bytes=32 * 1024 * 1024),
        cost_estimate=cost,
    )(x3, mats, corr, bias_s)

    return out.reshape(B, 1, H, W)


def kernel(x, weight, bias):
    return _spatial_attention(x, weight, bias)


# speedup vs baseline: 1.2706x; 1.2706x over previous
import functools

import jax
import jax.numpy as jnp
from jax.experimental import pallas as pl
from jax.experimental.pallas import tpu as pltpu


def _round_up(v, m):
    return ((v + m - 1) // m) * m


def _tree_reduce(vals, op):
    vals = list(vals)
    while len(vals) > 1:
        nxt = [op(vals[i], vals[i + 1]) for i in range(0, len(vals) - 1, 2)]
        if len(vals) % 2:
            nxt.append(vals[-1])
        vals = nxt
    return vals[0]


def _toeplitz(wrow, n):
    k = wrow.shape[0]
    p = k // 2
    m = jnp.zeros((n, n), jnp.float32)
    for d in range(-p, p + 1):
        m = m + wrow[d + p] * jnp.eye(n, n, k=-d, dtype=jnp.float32)
    return m


def _build_conv_mats(w0, w1c, wth, kk):
    p = kk // 2
    mats = jnp.zeros((5, 4 * wth, 2 * wth), jnp.float32)
    for dlt in range(-2, 3):
        for pp in (0, 1):
            for q in (0, 1):
                ki = 2 * dlt + pp - q + p
                if 0 <= ki < kk:
                    t0 = _toeplitz(w0[ki], wth)
                    t1 = _toeplitz(w1c[ki], wth)
                    r0 = pp * wth
                    c0 = q * wth
                    mats = mats.at[dlt + 2, r0:r0 + wth, c0:c0 + wth].add(t0)
                    mats = mats.at[dlt + 2, 2 * wth + r0:2 * wth + r0 + wth,
                                   c0:c0 + wth].add(t1)
    return mats


def _sa_body(x_ref, m_ref, corr_ref, s_ref, o_ref, pad_ref, *,
             C, HPAIR, W2):
    rows = 8
    cpi = min(4, C // rows)
    n_iter = C // (rows * cpi)
    step_c = rows * cpi

    for s in range(HPAIR):
        lane0 = s * W2

        def body(i, carry, _lane0=lane0):
            m, su = carry
            base = pl.multiple_of(i * step_c, step_c)
            cs = [x_ref[0, pl.ds(base + k * rows, rows), pl.ds(_lane0, W2)]
                  for k in range(cpi)]
            m1 = _tree_reduce(cs, jnp.maximum)
            s1 = _tree_reduce(cs, jnp.add)
            return jnp.maximum(m, m1), su + s1

        init = (jnp.full((rows, W2), -jnp.inf, jnp.float32),
                jnp.zeros((rows, W2), jnp.float32))
        mx, sm = jax.lax.fori_loop(0, n_iter, body, init, unroll=2)
        pad_ref[pl.ds(2 + s, 1), 0:W2] = jnp.max(mx, axis=0, keepdims=True)
        pad_ref[pl.ds(2 + s, 1), W2:2 * W2] = jnp.sum(sm, axis=0,
                                                      keepdims=True)

    blk = pad_ref[pl.ds(2, HPAIR), :]
    c0 = jnp.mean(blk[:, 0:W2])
    c1 = jnp.mean(blk[:, W2:2 * W2])
    lane = jax.lax.broadcasted_iota(jnp.int32, (HPAIR, 2 * W2), 1)
    offs = jnp.where(lane < W2, c0, c1)
    pad_ref[0:2, :] = jnp.zeros((2, 2 * W2), jnp.float32)
    pad_ref[pl.ds(2 + HPAIR, 2), :] = jnp.zeros((2, 2 * W2), jnp.float32)
    pad_ref[pl.ds(2, HPAIR), :] = blk - offs

    acc = None
    for dlt in range(5):
        win = pad_ref[pl.ds(dlt, HPAIR), :]
        mm = jnp.dot(win, m_ref[dlt], preferred_element_type=jnp.float32)
        acc = mm if acc is None else acc + mm

    z = acc + c0 * corr_ref[0] + c1 * corr_ref[1] + s_ref[0]
    o_ref[0, 0] = jax.nn.sigmoid(z).astype(o_ref.dtype)


def _spatial_attention(x, weight, bias):
    B, C, H, W = x.shape
    kk = weight.shape[2]
    p = kk // 2
    assert H % 2 == 0 and W == 64 and C % 8 == 0
    hpair = H // 2
    w2 = 2 * W

    x_flat = x.reshape(B, C, H * W)

    w0 = weight[0, 0].astype(jnp.float32)
    w1c = weight[0, 1].astype(jnp.float32) * (1.0 / C)
    mats = _build_conv_mats(w0, w1c, W, kk)

    hh = jnp.arange(H)[:, None] + jnp.arange(kk)[None, :] - p
    um = ((hh >= 0) & (hh < H)).astype(jnp.float32)
    wwv = jnp.arange(W)[:, None] + jnp.arange(kk)[None, :] - p
    vm = ((wwv >= 0) & (wwv < W)).astype(jnp.float32)
    s0 = um @ w0 @ vm.T
    s1 = um @ w1c @ vm.T
    corr = jnp.stack([s0.reshape(hpair, w2), s1.reshape(hpair, w2)])

    bias_s = bias.reshape(-1).astype(jnp.float32)

    pad_rows = _round_up(hpair + 4, 8)
    body = functools.partial(_sa_body, C=C, HPAIR=hpair, W2=w2)

    cost = pl.CostEstimate(
        flops=int(B * H * W * (2 * C + 4 * kk * kk + 4)),
        transcendentals=int(B * H * W),
        bytes_accessed=int(B * (C + 1) * H * W * 4 + mats.size * 4),
    )

    out = pl.pallas_call(
        body,
        out_shape=jax.ShapeDtypeStruct((B, 1, hpair, w2), x.dtype),
        grid=(B,),
        in_specs=[
            pl.BlockSpec((1, C, H * W), lambda b: (b, 0, 0)),
            pl.BlockSpec((5, 4 * W, w2), lambda b: (0, 0, 0)),
            pl.BlockSpec((2, hpair, w2), lambda b: (0, 0, 0)),
            pl.BlockSpec(memory_space=pltpu.MemorySpace.SMEM),
        ],
        out_specs=pl.BlockSpec((1, 1, hpair, w2), lambda b: (b, 0, 0, 0)),
        scratch_shapes=[
            pltpu.VMEM((pad_rows, 2 * w2), jnp.float32),
        ],
        compiler_params=pltpu.CompilerParams(
            dimension_semantics=("parallel",),
            vmem_limit_bytes=32 * 1024 * 1024),
        cost_estimate=cost,
    )(x_flat, mats, corr, bias_s)

    return out.reshape(B, 1, H, W)


def kernel(x, weight, bias):
    return _spatial_attention(x, weight, bias)
